# baseline (speedup 1.0000x reference)
import functools

import jax
import jax.numpy as jnp
from jax import lax
from jax.experimental import pallas as pl
from jax.experimental.pallas import tpu as pltpu
from jax.experimental.pallas import tpu_sc as plsc

_CHUNK = 128


@functools.lru_cache(maxsize=None)
def _make_kernel(B: int, V: int, D: int):
    info = plsc.get_sparse_core_info()
    NC, NS = info.num_cores, info.num_subcores
    NW = NC * NS
    rows_per_w = B // NW
    n_chunks = rows_per_w // _CHUNK
    mesh = plsc.VectorSubcoreMesh(core_axis_name="c", subcore_axis_name="s")

    @functools.partial(
        pl.kernel,
        out_type=jax.ShapeDtypeStruct((B, D), jnp.float32),
        mesh=mesh,
        scratch_types=[
            pltpu.VMEM((n_chunks, _CHUNK), jnp.int32),
            pltpu.VMEM((_CHUNK, D), jnp.float32),
            pltpu.VMEM((_CHUNK, D), jnp.float32),
            pltpu.SemaphoreType.DMA,
            pltpu.SemaphoreType.DMA,
        ],
    )
    def k(ids_hbm, table_hbm, out_hbm, idx_v, buf0, buf1, sem0, sem1):
        wid = lax.axis_index("s") * NC + lax.axis_index("c")
        base = wid * rows_per_w
        pltpu.sync_copy(ids_hbm.at[pl.ds(wid * n_chunks, n_chunks)], idx_v)
        bufs = (buf0, buf1)
        sems = (sem0, sem1)
        copies = [None] * n_chunks
        copies[0] = pltpu.async_copy(
            table_hbm.at[idx_v.at[0]], bufs[0], sems[0]
        )
        for c in range(n_chunks):
            if c + 1 < n_chunks:
                copies[c + 1] = pltpu.async_copy(
                    table_hbm.at[idx_v.at[c + 1]],
                    bufs[(c + 1) % 2],
                    sems[(c + 1) % 2],
                )
            copies[c].wait()
            pltpu.sync_copy(
                bufs[c % 2], out_hbm.at[pl.ds(base + c * _CHUNK, _CHUNK)]
            )

    return k


def kernel(ids, table):
    B = ids.size
    V, D = table.shape
    ids2d = ids.reshape(B // _CHUNK, _CHUNK).astype(jnp.int32)
    out = _make_kernel(B, V, D)(ids2d, table.astype(jnp.float32))
    return out.reshape(ids.shape + (D,))


# speedup vs baseline: 4.9895x; 4.9895x over previous
import functools

import jax
import jax.numpy as jnp
from jax import lax
from jax.experimental import pallas as pl
from jax.experimental.pallas import tpu as pltpu
from jax.experimental.pallas import tpu_sc as plsc

_CHUNK = 128
_L = 16


@functools.lru_cache(maxsize=None)
def _make_kernel(B: int, V: int, D: int):
    info = plsc.get_sparse_core_info()
    NC, NS = info.num_cores, info.num_subcores
    NW = NC * NS
    rows_per_w = B // NW
    n_chunks = rows_per_w // _CHUNK
    mesh = plsc.VectorSubcoreMesh(core_axis_name="c", subcore_axis_name="s")

    @functools.partial(
        pl.kernel,
        out_type=jax.ShapeDtypeStruct((B, D), jnp.float32),
        mesh=mesh,
        scratch_types=[
            pltpu.VMEM((rows_per_w,), jnp.int32),
            pltpu.VMEM((_CHUNK, D), jnp.float32),
            pltpu.VMEM((_CHUNK, D), jnp.float32),
            pltpu.SemaphoreType.DMA,
            pltpu.SemaphoreType.DMA,
        ],
    )
    def k(ids_hbm, table_hbm, out_hbm, idx_v, buf0, buf1, sem0, sem1):
        wid = lax.axis_index("s") * NC + lax.axis_index("c")
        base = wid * rows_per_w
        pltpu.sync_copy(ids_hbm.at[pl.ds(base, rows_per_w)], idx_v)
        off = wid * V
        for i in range(rows_per_w // _L):
            sl = pl.ds(i * _L, _L)
            idx_v[sl] = idx_v[sl] + off
        bufs = (buf0, buf1)
        sems = (sem0, sem1)
        copies = [None] * n_chunks
        copies[0] = pltpu.async_copy(
            table_hbm.at[idx_v.at[pl.ds(0, _CHUNK)]], bufs[0], sems[0]
        )
        for c in range(n_chunks):
            if c + 1 < n_chunks:
                copies[c + 1] = pltpu.async_copy(
                    table_hbm.at[idx_v.at[pl.ds((c + 1) * _CHUNK, _CHUNK)]],
                    bufs[(c + 1) % 2],
                    sems[(c + 1) % 2],
                )
            copies[c].wait()
            pltpu.sync_copy(
                bufs[c % 2], out_hbm.at[pl.ds(base + c * _CHUNK, _CHUNK)]
            )

    return k, NW


def kernel(ids, table):
    B = ids.size
    V, D = table.shape
    k, NW = _make_kernel(B, V, D)
    ids_flat = ids.reshape(B).astype(jnp.int32)
    table_rep = jnp.tile(table.astype(jnp.float32), (NW, 1))
    out = k(ids_flat, table_rep)
    return out.reshape(ids.shape + (D,))


# speedup vs baseline: 8.9176x; 1.7873x over previous
import functools

import jax
import jax.numpy as jnp
from jax import lax
from jax.experimental import pallas as pl
from jax.experimental.pallas import tpu as pltpu
from jax.experimental.pallas import tpu_sc as plsc

_CHUNK = 128
_L = 16
_NBUF = 4


@functools.lru_cache(maxsize=None)
def _make_kernel(B: int, V: int, D: int):
    info = plsc.get_sparse_core_info()
    NC, NS = info.num_cores, info.num_subcores
    NW = NC * NS
    rows_per_w = B // NW
    n_chunks = rows_per_w // _CHUNK
    mesh = plsc.VectorSubcoreMesh(core_axis_name="c", subcore_axis_name="s")

    @functools.partial(
        pl.kernel,
        out_type=jax.ShapeDtypeStruct((B, D), jnp.float32),
        mesh=mesh,
        scratch_types=[
            pltpu.VMEM_SHARED((NS * V, D), jnp.float32),
            pltpu.VMEM((rows_per_w,), jnp.int32),
            pltpu.VMEM((_NBUF, _CHUNK, D), jnp.float32),
            pltpu.SemaphoreType.DMA,
            [pltpu.SemaphoreType.DMA] * _NBUF,
            [pltpu.SemaphoreType.DMA] * _NBUF,
        ],
    )
    def k(ids_hbm, table_hbm, out_hbm, tab_sh, idx_v, bufs, sem_tab,
          gsems, wsems):
        cid = lax.axis_index("c")
        sid = lax.axis_index("s")
        wid = sid * NC + cid
        base = wid * rows_per_w
        pltpu.sync_copy(ids_hbm.at[pl.ds(base, rows_per_w)], idx_v)
        @pl.when(sid == 0)
        def _stage():
            for r in range(NS):
                pltpu.async_copy(
                    table_hbm, tab_sh.at[pl.ds(r * V, V)], sem_tab
                )
            for r in range(NS):
                pltpu.make_async_copy(
                    table_hbm, tab_sh.at[pl.ds(r * V, V)], sem_tab
                ).wait()

        off = sid * V
        for i in range(rows_per_w // _L):
            sl = pl.ds(i * _L, _L)
            idx_v[sl] = idx_v[sl] + off
        plsc.subcore_barrier()

        def gather(c):
            return pltpu.async_copy(
                tab_sh.at[idx_v.at[pl.ds(c * _CHUNK, _CHUNK)]],
                bufs.at[c % _NBUF],
                gsems[c % _NBUF],
            )

        gcp = [None] * n_chunks
        wcp = [None] * n_chunks
        for c in range(min(_NBUF, n_chunks)):
            gcp[c] = gather(c)
        for c in range(n_chunks):
            if c >= _NBUF:
                wcp[c - _NBUF].wait()
                gcp[c] = gather(c)
            gcp[c].wait()
            wcp[c] = pltpu.async_copy(
                bufs.at[c % _NBUF],
                out_hbm.at[pl.ds(base + c * _CHUNK, _CHUNK)],
                wsems[c % _NBUF],
            )
        for c in range(max(0, n_chunks - _NBUF), n_chunks):
            wcp[c].wait()

    return k


def kernel(ids, table):
    B = ids.size
    V, D = table.shape
    k = _make_kernel(B, V, D)
    out = k(ids.reshape(B).astype(jnp.int32), table.astype(jnp.float32))
    return out.reshape(ids.shape + (D,))


# speedup vs baseline: 9.1806x; 1.0295x over previous
import functools

import jax
import jax.numpy as jnp
from jax import lax
from jax.experimental import pallas as pl
from jax.experimental.pallas import tpu as pltpu
from jax.experimental.pallas import tpu_sc as plsc

_CHUNK = 128
_L = 16
_NBUF = 6


@functools.lru_cache(maxsize=None)
def _make_kernel(B: int, V: int, D: int):
    info = plsc.get_sparse_core_info()
    NC, NS = info.num_cores, info.num_subcores
    NW = NC * NS
    rows_per_w = B // NW
    n_chunks = rows_per_w // _CHUNK
    mesh = plsc.VectorSubcoreMesh(core_axis_name="c", subcore_axis_name="s")

    @functools.partial(
        pl.kernel,
        out_type=jax.ShapeDtypeStruct((B, D), jnp.float32),
        mesh=mesh,
        scratch_types=[
            pltpu.VMEM_SHARED((NS * V, D), jnp.float32),
            pltpu.VMEM((rows_per_w,), jnp.int32),
            pltpu.VMEM((_NBUF, _CHUNK, D), jnp.float32),
            pltpu.SemaphoreType.DMA,
            pltpu.SemaphoreType.DMA,
            [pltpu.SemaphoreType.DMA] * _NBUF,
            [pltpu.SemaphoreType.DMA] * _NBUF,
        ],
    )
    def k(ids_hbm, table_hbm, out_hbm, tab_sh, idx_v, bufs, sem_tab,
          sem_ids, gsems, wsems):
        cid = lax.axis_index("c")
        sid = lax.axis_index("s")
        wid = sid * NC + cid
        base = wid * rows_per_w
        ids_cp = pltpu.async_copy(
            ids_hbm.at[pl.ds(base, rows_per_w)], idx_v, sem_ids
        )
        @pl.when(sid == 0)
        def _stage():
            for r in range(NS):
                pltpu.async_copy(
                    table_hbm, tab_sh.at[pl.ds(r * V, V)], sem_tab
                )
            for r in range(NS):
                pltpu.make_async_copy(
                    table_hbm, tab_sh.at[pl.ds(r * V, V)], sem_tab
                ).wait()

        ids_cp.wait()
        off = sid * V
        for i in range(rows_per_w // _L):
            sl = pl.ds(i * _L, _L)
            idx_v[sl] = idx_v[sl] + off
        plsc.subcore_barrier()

        def gather(c):
            return pltpu.async_copy(
                tab_sh.at[idx_v.at[pl.ds(c * _CHUNK, _CHUNK)]],
                bufs.at[c % _NBUF],
                gsems[c % _NBUF],
            )

        gcp = [None] * n_chunks
        wcp = [None] * n_chunks
        for c in range(min(_NBUF, n_chunks)):
            gcp[c] = gather(c)
        for c in range(n_chunks):
            if c >= _NBUF:
                wcp[c - _NBUF].wait()
                gcp[c] = gather(c)
            gcp[c].wait()
            wcp[c] = pltpu.async_copy(
                bufs.at[c % _NBUF],
                out_hbm.at[pl.ds(base + c * _CHUNK, _CHUNK)],
                wsems[c % _NBUF],
            )
        for c in range(max(0, n_chunks - _NBUF), n_chunks):
            wcp[c].wait()

    return k


def kernel(ids, table):
    B = ids.size
    V, D = table.shape
    k = _make_kernel(B, V, D)
    out = k(ids.reshape(B).astype(jnp.int32), table.astype(jnp.float32))
    return out.reshape(ids.shape + (D,))


# speedup vs baseline: 9.2027x; 1.0024x over previous
import functools

import jax
import jax.numpy as jnp
from jax import lax
from jax.experimental import pallas as pl
from jax.experimental.pallas import tpu as pltpu
from jax.experimental.pallas import tpu_sc as plsc

_CHUNK = 128
_L = 16
_NBUF = 6


@functools.lru_cache(maxsize=None)
def _make_kernel(NB: int, S: int, V: int, D: int):
    info = plsc.get_sparse_core_info()
    NC, NS = info.num_cores, info.num_subcores
    NW = NC * NS
    rows_per_w = (NB * S) // NW
    n_chunks = rows_per_w // _CHUNK
    wpb = S // rows_per_w
    mesh = plsc.VectorSubcoreMesh(core_axis_name="c", subcore_axis_name="s")

    @functools.partial(
        pl.kernel,
        out_type=jax.ShapeDtypeStruct((NB, S, D), jnp.float32),
        mesh=mesh,
        scratch_types=[
            pltpu.VMEM_SHARED((NS * V, D), jnp.float32),
            pltpu.VMEM((rows_per_w,), jnp.int32),
            pltpu.VMEM((_NBUF, _CHUNK, D), jnp.float32),
            pltpu.SemaphoreType.DMA,
            pltpu.SemaphoreType.DMA,
            [pltpu.SemaphoreType.DMA] * _NBUF,
            [pltpu.SemaphoreType.DMA] * _NBUF,
        ],
    )
    def k(ids_hbm, table_hbm, out_hbm, tab_sh, idx_v, bufs, sem_tab,
          sem_ids, gsems, wsems):
        cid = lax.axis_index("c")
        sid = lax.axis_index("s")
        wid = sid * NC + cid
        b = wid // wpb
        col = (wid % wpb) * rows_per_w
        ids_cp = pltpu.async_copy(
            ids_hbm.at[b, pl.ds(col, rows_per_w)], idx_v, sem_ids
        )
        @pl.when(sid == 0)
        def _stage():
            for r in range(NS):
                pltpu.async_copy(
                    table_hbm, tab_sh.at[pl.ds(r * V, V)], sem_tab
                )
            for r in range(NS):
                pltpu.make_async_copy(
                    table_hbm, tab_sh.at[pl.ds(r * V, V)], sem_tab
                ).wait()

        ids_cp.wait()
        off = sid * V
        for i in range(rows_per_w // _L):
            sl = pl.ds(i * _L, _L)
            idx_v[sl] = idx_v[sl] + off
        plsc.subcore_barrier()

        def gather(c):
            return pltpu.async_copy(
                tab_sh.at[idx_v.at[pl.ds(c * _CHUNK, _CHUNK)]],
                bufs.at[c % _NBUF],
                gsems[c % _NBUF],
            )

        gcp = [None] * n_chunks
        wcp = [None] * n_chunks
        for c in range(min(_NBUF, n_chunks)):
            gcp[c] = gather(c)
        for c in range(n_chunks):
            if c >= _NBUF:
                wcp[c - _NBUF].wait()
                gcp[c] = gather(c)
            gcp[c].wait()
            wcp[c] = pltpu.async_copy(
                bufs.at[c % _NBUF],
                out_hbm.at[b, pl.ds(col + c * _CHUNK, _CHUNK)],
                wsems[c % _NBUF],
            )
        for c in range(max(0, n_chunks - _NBUF), n_chunks):
            wcp[c].wait()

    return k


def kernel(ids, table):
    NB, S = ids.shape
    V, D = table.shape
    k = _make_kernel(NB, S, V, D)
    return k(ids.astype(jnp.int32), table.astype(jnp.float32))
